# baseline (speedup 1.0000x reference)
import functools

import jax
import jax.numpy as jnp
from jax import lax
from jax.experimental import pallas as pl
from jax.experimental.pallas import tpu as pltpu
from jax.experimental.pallas import tpu_sc as plsc

_B = 2
_N = 4096
_CH = 128
_KCH = 64
_NS = 16
_K = 8
_QP = 512
_QD = 256
_QM = 256
_GCHUNK = 256
_NW = 32


def _lrelu(x):
    return jnp.where(x > 0, x, 0.1 * x)


def _prep_body(k1t_r, k2t_r, f1_r, f2_r, x1_r, x2_r,
               W11_r, b11_r, W22_r, b22_r, Wpos_r, bpos_r,
               k1n_r, k2n_r, a1d1_r, a2d1_r, a1d2_r, a2d2_r):
    k1 = k1t_r[...]
    k2 = k2t_r[...]
    k1n_r[...] = k1 / jnp.sqrt(jnp.sum(k1 * k1, axis=-1, keepdims=True) + 1e-08)
    k2n_r[...] = k2 / jnp.sqrt(jnp.sum(k2 * k2, axis=-1, keepdims=True) + 1e-08)

    f1 = f1_r[...]
    f2 = f2_r[...]
    x1 = x1_r[...]
    x2 = x2_r[...]
    W11 = W11_r[...]
    W22 = W22_r[...]
    Wpos = Wpos_r[...]
    b11 = b11_r[...]
    b22 = b22_r[...]
    bpos = bpos_r[...]

    dn = (((1,), (1,)), ((), ()))
    t1 = lax.dot_general(f1.T, W11, dn) + b11
    t2 = lax.dot_general(f2.T, W22, dn) + b22
    t3 = lax.dot_general(f2.T, W11, dn) + b11
    t4 = lax.dot_general(f1.T, W22, dn) + b22
    p1 = lax.dot_general(x1.T, Wpos, dn)
    p2 = lax.dot_general(x2.T, Wpos, dn)
    a1d1_r[...] = t1 - p1 + bpos
    a2d1_r[...] = t2 + p2
    a1d2_r[...] = t3 - p2 + bpos
    a2d2_r[...] = t4 + p1


def _run_prep(k1t, k2t, feat1, feat2, pc1, pc2, W11, b11, W22, b22, Wpos, bpos):
    nb = _N // _QP
    qspec = pl.BlockSpec((None, _QP, _KCH), lambda b, n: (b, n, 0))
    fspec = pl.BlockSpec((None, _CH, _QP), lambda b, n: (b, 0, n))
    xspec = pl.BlockSpec((None, 3, _QP), lambda b, n: (b, 0, n))
    wspec = pl.BlockSpec((_CH, _CH), lambda b, n: (0, 0))
    wpspec = pl.BlockSpec((_CH, 3), lambda b, n: (0, 0))
    bspec = pl.BlockSpec((1, _CH), lambda b, n: (0, 0))
    ospec_k = pl.BlockSpec((None, _QP, _KCH), lambda b, n: (b, n, 0))
    ospec_a = pl.BlockSpec((None, _QP, _CH), lambda b, n: (b, n, 0))
    sd_k = jax.ShapeDtypeStruct((_B, _N, _KCH), jnp.float32)
    sd_a = jax.ShapeDtypeStruct((_B, _N, _CH), jnp.float32)
    return pl.pallas_call(
        _prep_body,
        grid=(_B, nb),
        in_specs=[qspec, qspec, fspec, fspec, xspec, xspec,
                  wspec, bspec, wspec, bspec, wpspec, bspec],
        out_specs=[ospec_k, ospec_k, ospec_a, ospec_a, ospec_a, ospec_a],
        out_shape=[sd_k, sd_k, sd_a, sd_a, sd_a, sd_a],
    )(k1t, k2t, feat1, feat2, pc1, pc2, W11, b11, W22, b22, Wpos, bpos)


def _top8_cols(d, base):
    q = d.shape[0]
    iota = lax.broadcasted_iota(jnp.int32, (q, _N), 1)
    big_i = jnp.int32(_N)
    inf = jnp.float32(jnp.inf)
    cols = []
    for _ in range(_K):
        m = jnp.min(d, axis=1, keepdims=True)
        i = jnp.min(jnp.where(d == m, iota, big_i), axis=1)
        cols.append(i + base)
        d = jnp.where(iota == i[:, None], inf, d)
    return cols


def _dist_body(dir_off, qk_r, ck_r, qx_r, cx_r, idx_r):
    b = pl.program_id(0)
    base = (dir_off + b) * _N

    qk = qk_r[...]
    ck = ck_r[...]
    dn = (((1,), (0,)), ((), ()))
    dcos = 1.0 - lax.dot_general(qk, ck, dn)

    qx = qx_r[...]
    cx = cx_r[...]
    dsq = -2.0 * lax.dot_general(qx, cx, dn)
    dsq = dsq + jnp.sum(qx * qx, axis=1, keepdims=True)
    dsq = dsq + jnp.sum(cx * cx, axis=0, keepdims=True)

    cols = _top8_cols(dcos, base) + _top8_cols(dsq, base)
    idx_r[...] = jnp.concatenate([c[:, None] for c in cols], axis=1)


def _run_dist(qk, ck, qxt, cx, dir_off):
    nb = _N // _QD
    return pl.pallas_call(
        functools.partial(_dist_body, dir_off),
        grid=(_B, nb),
        in_specs=[
            pl.BlockSpec((None, _QD, _KCH), lambda b, n: (b, n, 0)),
            pl.BlockSpec((None, _KCH, _N), lambda b, n: (b, 0, 0)),
            pl.BlockSpec((None, _QD, 3), lambda b, n: (b, n, 0)),
            pl.BlockSpec((None, 3, _N), lambda b, n: (b, 0, 0)),
        ],
        out_specs=pl.BlockSpec((None, _QD, _NS), lambda b, n: (b, n, 0)),
        out_shape=jax.ShapeDtypeStruct((_B, _N, _NS), jnp.int32),
    )(qk, ck, qxt, cx)


_NROWS = _B * _NS * _N
_RPW = _NROWS // _NW
_NCHG = _RPW // _GCHUNK


def _sc_gather(table, idxflat):
    mesh = plsc.VectorSubcoreMesh(core_axis_name="c", subcore_axis_name="s")

    @functools.partial(
        pl.kernel, mesh=mesh,
        out_type=jax.ShapeDtypeStruct((_NROWS, _CH), jnp.float32),
        scratch_types=[
            pltpu.VMEM((_RPW,), jnp.int32),
            pltpu.VMEM((_GCHUNK, _CH), jnp.float32),
            pltpu.VMEM((_GCHUNK, _CH), jnp.float32),
            pltpu.SemaphoreType.DMA,
            pltpu.SemaphoreType.DMA,
        ],
    )
    def k(table_hbm, idx_hbm, out_hbm, idx_v, buf0, buf1, sem0, sem1):
        wid = lax.axis_index("s") * 2 + lax.axis_index("c")
        base = wid * _RPW
        pltpu.sync_copy(idx_hbm.at[pl.ds(base, _RPW)], idx_v)

        def gcopy(c, buf, sem):
            return pltpu.make_async_copy(
                table_hbm.at[idx_v.at[pl.ds(c * _GCHUNK, _GCHUNK)]], buf, sem)

        gcopy(0, buf0, sem0).start()

        def body(j, carry):
            c0 = 2 * j
            gcopy(c0, buf0, sem0).wait()
            gcopy(c0 + 1, buf1, sem1).start()
            pltpu.sync_copy(buf0, out_hbm.at[pl.ds(base + c0 * _GCHUNK, _GCHUNK)])
            gcopy(c0 + 1, buf1, sem1).wait()

            @pl.when(c0 + 2 < _NCHG)
            def _():
                gcopy(c0 + 2, buf0, sem0).start()

            pltpu.sync_copy(buf1,
                            out_hbm.at[pl.ds(base + (c0 + 1) * _GCHUNK, _GCHUNK)])
            return carry

        lax.fori_loop(0, _NCHG // 2, body, jnp.int32(0))

    return k(table, idxflat)


def _mlp_body(g_r, a1_r, Wm1_r, bm1_r, Wm2_r, bm2_r, out_r):
    a1 = a1_r[...]
    Wm1 = Wm1_r[...]
    Wm2 = Wm2_r[...]
    bm1 = bm1_r[...]
    bm2t = bm2_r[...]
    dn_nt = (((1,), (1,)), ((), ()))
    acc = None
    for s in range(_NS):
        x = _lrelu(g_r[s] + a1)
        h = _lrelu(lax.dot_general(x, Wm1, dn_nt) + bm1)
        y = _lrelu(lax.dot_general(Wm2, h, dn_nt) + bm2t)
        acc = y if acc is None else jnp.maximum(acc, y)
    out_r[...] = acc


def _run_mlp(g5, a1s, Wm1, bm1, Wm2, bm2t):
    nb = _N // _QM
    return pl.pallas_call(
        _mlp_body,
        grid=(2, _B, nb),
        in_specs=[
            pl.BlockSpec((None, None, _NS, _QM, _CH), lambda d, b, n: (d, b, 0, n, 0)),
            pl.BlockSpec((None, None, _QM, _CH), lambda d, b, n: (d, b, n, 0)),
            pl.BlockSpec((_CH, _CH), lambda d, b, n: (0, 0)),
            pl.BlockSpec((1, _CH), lambda d, b, n: (0, 0)),
            pl.BlockSpec((_CH, _CH), lambda d, b, n: (0, 0)),
            pl.BlockSpec((_CH, 1), lambda d, b, n: (0, 0)),
        ],
        out_specs=pl.BlockSpec((None, None, _CH, _QM), lambda d, b, n: (d, b, 0, n)),
        out_shape=jax.ShapeDtypeStruct((2, _B, _CH, _N), jnp.float32),
    )(g5, a1s, Wm1, bm1, Wm2, bm2t)


def kernel(pc1, pc2, feat1, feat2, knn1, knn2, W11, b11, W22, b22, Wpos, bpos, Wm1, bm1, Wm2, bm2):
    k1t = knn1.transpose(0, 2, 1)
    k2t = knn2.transpose(0, 2, 1)
    x1t = pc1.transpose(0, 2, 1)
    x2t = pc2.transpose(0, 2, 1)
    b11r = b11.reshape(1, _CH)
    b22r = b22.reshape(1, _CH)
    bposr = bpos.reshape(1, _CH)

    k1n, k2n, a1d1, a2d1, a1d2, a2d2 = _run_prep(
        k1t, k2t, feat1, feat2, pc1, pc2, W11, b11r, W22, b22r, Wpos, bposr)

    k1nT = k1n.transpose(0, 2, 1)
    k2nT = k2n.transpose(0, 2, 1)
    table = jnp.concatenate([a2d1, a2d2], axis=0).reshape(2 * _B * _N, _CH)
    idx1 = _run_dist(k1n, k2nT, x1t, pc2, 0)
    g1 = _sc_gather(table, idx1.transpose(0, 2, 1).reshape(_NROWS))
    idx2 = _run_dist(k2n, k1nT, x2t, pc1, _B)
    g2 = _sc_gather(table, idx2.transpose(0, 2, 1).reshape(_NROWS))

    g5 = jnp.stack([g1.reshape(_B, _NS, _N, _CH),
                    g2.reshape(_B, _NS, _N, _CH)], axis=0)

    a1s = jnp.stack([a1d1, a1d2], axis=0)
    out = _run_mlp(g5, a1s, Wm1, bm1.reshape(1, _CH), Wm2, bm2.reshape(_CH, 1))
    return (out[0], out[1])


# speedup vs baseline: 1.0416x; 1.0416x over previous
import functools

import jax
import jax.numpy as jnp
from jax import lax
from jax.experimental import pallas as pl
from jax.experimental.pallas import tpu as pltpu
from jax.experimental.pallas import tpu_sc as plsc

_B = 2
_N = 4096
_CH = 128
_KCH = 64
_NS = 16
_K = 8
_QP = 512
_QD = 256
_QM = 256
_GCHUNK = 256
_NW = 32


def _lrelu(x):
    return jnp.where(x > 0, x, 0.1 * x)


def _prep_body(k1t_r, k2t_r, f1_r, f2_r, x1_r, x2_r,
               W11_r, b11_r, W22_r, b22_r, Wpos_r, bpos_r,
               k1n_r, k2n_r, a1d1_r, a2d1_r, a1d2_r, a2d2_r):
    k1 = k1t_r[...]
    k2 = k2t_r[...]
    k1n_r[...] = k1 / jnp.sqrt(jnp.sum(k1 * k1, axis=-1, keepdims=True) + 1e-08)
    k2n_r[...] = k2 / jnp.sqrt(jnp.sum(k2 * k2, axis=-1, keepdims=True) + 1e-08)

    f1 = f1_r[...]
    f2 = f2_r[...]
    x1 = x1_r[...]
    x2 = x2_r[...]
    W11 = W11_r[...]
    W22 = W22_r[...]
    Wpos = Wpos_r[...]
    b11 = b11_r[...]
    b22 = b22_r[...]
    bpos = bpos_r[...]

    dn = (((1,), (1,)), ((), ()))
    t1 = lax.dot_general(f1.T, W11, dn) + b11
    t2 = lax.dot_general(f2.T, W22, dn) + b22
    t3 = lax.dot_general(f2.T, W11, dn) + b11
    t4 = lax.dot_general(f1.T, W22, dn) + b22
    p1 = lax.dot_general(x1.T, Wpos, dn)
    p2 = lax.dot_general(x2.T, Wpos, dn)
    a1d1_r[...] = t1 - p1 + bpos
    a2d1_r[...] = t2 + p2
    a1d2_r[...] = t3 - p2 + bpos
    a2d2_r[...] = t4 + p1


def _run_prep(k1t, k2t, feat1, feat2, pc1, pc2, W11, b11, W22, b22, Wpos, bpos):
    nb = _N // _QP
    qspec = pl.BlockSpec((None, _QP, _KCH), lambda b, n: (b, n, 0))
    fspec = pl.BlockSpec((None, _CH, _QP), lambda b, n: (b, 0, n))
    xspec = pl.BlockSpec((None, 3, _QP), lambda b, n: (b, 0, n))
    wspec = pl.BlockSpec((_CH, _CH), lambda b, n: (0, 0))
    wpspec = pl.BlockSpec((_CH, 3), lambda b, n: (0, 0))
    bspec = pl.BlockSpec((1, _CH), lambda b, n: (0, 0))
    ospec_k = pl.BlockSpec((None, _QP, _KCH), lambda b, n: (b, n, 0))
    ospec_a = pl.BlockSpec((None, _QP, _CH), lambda b, n: (b, n, 0))
    sd_k = jax.ShapeDtypeStruct((_B, _N, _KCH), jnp.float32)
    sd_a = jax.ShapeDtypeStruct((_B, _N, _CH), jnp.float32)
    return pl.pallas_call(
        _prep_body,
        grid=(_B, nb),
        in_specs=[qspec, qspec, fspec, fspec, xspec, xspec,
                  wspec, bspec, wspec, bspec, wpspec, bspec],
        out_specs=[ospec_k, ospec_k, ospec_a, ospec_a, ospec_a, ospec_a],
        out_shape=[sd_k, sd_k, sd_a, sd_a, sd_a, sd_a],
    )(k1t, k2t, feat1, feat2, pc1, pc2, W11, b11, W22, b22, Wpos, bpos)


def _top8_cols(d, base):
    q = d.shape[0]
    iota = lax.broadcasted_iota(jnp.int32, (q, _N), 1)
    big_i = jnp.int32(_N)
    inf = jnp.float32(jnp.inf)
    cols = []
    for _ in range(_K):
        m = jnp.min(d, axis=1, keepdims=True)
        i = jnp.min(jnp.where(d == m, iota, big_i), axis=1)
        cols.append(i + base)
        d = jnp.where(iota == i[:, None], inf, d)
    return cols


def _dist_body(dir_off, qk_r, ck_r, qx_r, cx_r, idx_r):
    b = pl.program_id(0)
    base = (dir_off + b) * _N

    qk = qk_r[...]
    ck = ck_r[...]
    dn = (((1,), (0,)), ((), ()))
    dcos = 1.0 - lax.dot_general(qk, ck, dn)

    qx = qx_r[...]
    cx = cx_r[...]
    dsq = -2.0 * lax.dot_general(qx, cx, dn)
    dsq = dsq + jnp.sum(qx * qx, axis=1, keepdims=True)
    dsq = dsq + jnp.sum(cx * cx, axis=0, keepdims=True)

    cols = _top8_cols(dcos, base) + _top8_cols(dsq, base)
    idx_r[...] = jnp.concatenate([c[:, None] for c in cols], axis=1)


def _run_dist(qk, ck, qxt, cx, dir_off):
    nb = _N // _QD
    return pl.pallas_call(
        functools.partial(_dist_body, dir_off),
        grid=(_B, nb),
        in_specs=[
            pl.BlockSpec((None, _QD, _KCH), lambda b, n: (b, n, 0)),
            pl.BlockSpec((None, _KCH, _N), lambda b, n: (b, 0, 0)),
            pl.BlockSpec((None, _QD, 3), lambda b, n: (b, n, 0)),
            pl.BlockSpec((None, 3, _N), lambda b, n: (b, 0, 0)),
        ],
        out_specs=pl.BlockSpec((None, _QD, _NS), lambda b, n: (b, n, 0)),
        out_shape=jax.ShapeDtypeStruct((_B, _N, _NS), jnp.int32),
    )(qk, ck, qxt, cx)


_NROWS = 2 * _B * _NS * _N
_RPW = _NROWS // _NW
_NCHG = _RPW // _GCHUNK


def _sc_gather(table, idxflat):
    mesh = plsc.VectorSubcoreMesh(core_axis_name="c", subcore_axis_name="s")

    @functools.partial(
        pl.kernel, mesh=mesh,
        out_type=jax.ShapeDtypeStruct((_NROWS, _CH), jnp.float32),
        scratch_types=[
            pltpu.VMEM((_RPW,), jnp.int32),
            pltpu.VMEM((_GCHUNK, _CH), jnp.float32),
            pltpu.VMEM((_GCHUNK, _CH), jnp.float32),
            pltpu.SemaphoreType.DMA,
            pltpu.SemaphoreType.DMA,
        ],
    )
    def k(table_hbm, idx_hbm, out_hbm, idx_v, buf0, buf1, sem0, sem1):
        wid = lax.axis_index("s") * 2 + lax.axis_index("c")
        base = wid * _RPW
        pltpu.sync_copy(idx_hbm.at[pl.ds(base, _RPW)], idx_v)

        def gcopy(c, buf, sem):
            return pltpu.make_async_copy(
                table_hbm.at[idx_v.at[pl.ds(c * _GCHUNK, _GCHUNK)]], buf, sem)

        gcopy(0, buf0, sem0).start()

        def body(j, carry):
            c0 = 2 * j
            gcopy(c0, buf0, sem0).wait()
            gcopy(c0 + 1, buf1, sem1).start()
            pltpu.sync_copy(buf0, out_hbm.at[pl.ds(base + c0 * _GCHUNK, _GCHUNK)])
            gcopy(c0 + 1, buf1, sem1).wait()

            @pl.when(c0 + 2 < _NCHG)
            def _():
                gcopy(c0 + 2, buf0, sem0).start()

            pltpu.sync_copy(buf1,
                            out_hbm.at[pl.ds(base + (c0 + 1) * _GCHUNK, _GCHUNK)])
            return carry

        lax.fori_loop(0, _NCHG // 2, body, jnp.int32(0))

    return k(table, idxflat)


def _mlp_body(g_r, a1_r, Wm1_r, bm1_r, Wm2_r, bm2_r, out_r):
    a1 = a1_r[...]
    Wm1 = Wm1_r[...]
    Wm2 = Wm2_r[...]
    bm1 = bm1_r[...]
    bm2t = bm2_r[...]
    dn_nt = (((1,), (1,)), ((), ()))
    acc = None
    for s in range(_NS):
        x = _lrelu(g_r[s] + a1)
        h = _lrelu(lax.dot_general(x, Wm1, dn_nt) + bm1)
        y = _lrelu(lax.dot_general(Wm2, h, dn_nt) + bm2t)
        acc = y if acc is None else jnp.maximum(acc, y)
    out_r[...] = acc


def _run_mlp(g5, a1s, Wm1, bm1, Wm2, bm2t):
    nb = _N // _QM
    return pl.pallas_call(
        _mlp_body,
        grid=(2, _B, nb),
        in_specs=[
            pl.BlockSpec((None, None, _NS, _QM, _CH), lambda d, b, n: (d, b, 0, n, 0)),
            pl.BlockSpec((None, None, _QM, _CH), lambda d, b, n: (d, b, n, 0)),
            pl.BlockSpec((_CH, _CH), lambda d, b, n: (0, 0)),
            pl.BlockSpec((1, _CH), lambda d, b, n: (0, 0)),
            pl.BlockSpec((_CH, _CH), lambda d, b, n: (0, 0)),
            pl.BlockSpec((_CH, 1), lambda d, b, n: (0, 0)),
        ],
        out_specs=pl.BlockSpec((None, None, _CH, _QM), lambda d, b, n: (d, b, 0, n)),
        out_shape=jax.ShapeDtypeStruct((2, _B, _CH, _N), jnp.float32),
    )(g5, a1s, Wm1, bm1, Wm2, bm2t)


def kernel(pc1, pc2, feat1, feat2, knn1, knn2, W11, b11, W22, b22, Wpos, bpos, Wm1, bm1, Wm2, bm2):
    k1t = knn1.transpose(0, 2, 1)
    k2t = knn2.transpose(0, 2, 1)
    x1t = pc1.transpose(0, 2, 1)
    x2t = pc2.transpose(0, 2, 1)
    b11r = b11.reshape(1, _CH)
    b22r = b22.reshape(1, _CH)
    bposr = bpos.reshape(1, _CH)

    k1n, k2n, a1d1, a2d1, a1d2, a2d2 = _run_prep(
        k1t, k2t, feat1, feat2, pc1, pc2, W11, b11r, W22, b22r, Wpos, bposr)

    k1nT = k1n.transpose(0, 2, 1)
    k2nT = k2n.transpose(0, 2, 1)
    table = jnp.concatenate([a2d1, a2d2], axis=0).reshape(2 * _B * _N, _CH)
    idx1 = _run_dist(k1n, k2nT, x1t, pc2, 0)
    idx2 = _run_dist(k2n, k1nT, x2t, pc1, _B)
    idxs = jnp.stack([idx1, idx2], axis=0)
    idxflat = idxs.transpose(0, 1, 3, 2).reshape(_NROWS)

    g = _sc_gather(table, idxflat)
    g5 = g.reshape(2, _B, _NS, _N, _CH)

    a1s = jnp.stack([a1d1, a1d2], axis=0)
    out = _run_mlp(g5, a1s, Wm1, bm1.reshape(1, _CH), Wm2, bm2.reshape(_CH, 1))
    return (out[0], out[1])
